# baseline (speedup 1.0000x reference)
import functools

import jax
import jax.numpy as jnp
from jax import lax
from jax.experimental import pallas as pl
from jax.experimental.pallas import tpu as pltpu
from jax.experimental.pallas import tpu_sc as plsc

_NC = 2
_NS = 16
_NW = _NC * _NS
_L = 16
_NBUF = 4


def _sc_gather_add_t(xT, labelsT, table):
    t_dim, d, b = xT.shape
    bw = b // _NW
    assert bw * _NW == b and bw % _L == 0 and bw <= 128

    mesh = plsc.VectorSubcoreMesh(
        core_axis_name="c", subcore_axis_name="s",
        num_cores=_NC, num_subcores=_NS)

    @functools.partial(
        pl.kernel,
        out_type=jax.ShapeDtypeStruct((t_dim, d, b), jnp.float32),
        mesh=mesh,
        compiler_params=pltpu.CompilerParams(use_tc_tiling_on_sc=False, needs_layout_passes=False),
        scratch_types=(
            [pltpu.VMEM((bw,), jnp.int32) for _ in range(_NBUF)]
            + [pltpu.VMEM((bw, d), jnp.float32) for _ in range(_NBUF)]
            + [pltpu.VMEM((d, bw), jnp.float32) for _ in range(_NBUF)]
            + [pltpu.SemaphoreType.DMA for _ in range(4 * _NBUF)]
        ),
    )
    def k(x_hbm, lab_hbm, tab_hbm, out_hbm, *scr):
        idx_v = scr[:_NBUF]
        g_v = scr[_NBUF:2 * _NBUF]
        xo_v = scr[2 * _NBUF:3 * _NBUF]
        s_ix = scr[3 * _NBUF:4 * _NBUF]
        s_x = scr[4 * _NBUF:5 * _NBUF]
        s_g = scr[5 * _NBUF:6 * _NBUF]
        s_o = scr[6 * _NBUF:7 * _NBUF]
        wid = lax.axis_index("s") * _NC + lax.axis_index("c")
        b0 = wid * bw
        lane = lax.iota(jnp.int32, 16)

        def load(t, p):
            pltpu.async_copy(lab_hbm.at[t, pl.ds(b0, bw)], idx_v[p], s_ix[p])
            pltpu.async_copy(x_hbm.at[t, :, pl.ds(b0, bw)], xo_v[p], s_x[p])

        def wait_idx(p):
            pltpu.make_async_copy(
                lab_hbm.at[0, pl.ds(b0, bw)], idx_v[p], s_ix[p]).wait()

        def wait_x(p):
            pltpu.make_async_copy(
                x_hbm.at[0, :, pl.ds(b0, bw)], xo_v[p], s_x[p]).wait()

        def gather(p):
            pltpu.async_copy(tab_hbm.at[idx_v[p]], g_v[p], s_g[p])

        def wait_gather(p):
            pltpu.make_async_copy(tab_hbm.at[idx_v[p]], g_v[p], s_g[p]).wait()

        def store(t, p):
            pltpu.async_copy(xo_v[p], out_hbm.at[t, :, pl.ds(b0, bw)], s_o[p])

        def wait_store(p):
            pltpu.make_async_copy(
                xo_v[p], out_hbm.at[0, :, pl.ds(b0, bw)], s_o[p]).wait()

        def compute(p):
            g = g_v[p]
            xo = xo_v[p]
            for bg in range(bw // _L):
                bidx = bg * _L + lane

                def drow(dd, c2, bidx=bidx, g=g, xo=xo, bg=bg):
                    vals = plsc.load_gather(
                        g, [bidx, jnp.full((16,), 0, jnp.int32) + dd])
                    plsc.addupdate(xo.at[dd, pl.ds(bg * _L, _L)], vals)
                    return c2

                lax.fori_loop(0, d, drow, 0, unroll=4)

        def step(t, p, drain_store, prefetch, fire_gather):
            q2 = (p + 2) % _NBUF
            if drain_store:
                wait_store(q2)
            if prefetch:
                load(t + 2, q2)
            q1 = (p + 1) % _NBUF
            if fire_gather:
                wait_idx(q1)
                gather(q1)
            wait_gather(p)
            wait_x(p)
            compute(p)
            store(t, p)

        nsteps = t_dim
        rounds = nsteps // _NBUF
        load(0, 0)
        load(1, 1)
        wait_idx(0)
        gather(0)

        for p in range(_NBUF):
            step(p, p, drain_store=(p >= 2), prefetch=True, fire_gather=True)

        def round_body(gr, carry):
            t0 = gr * _NBUF
            for p in range(_NBUF):
                step(t0 + p, p, drain_store=True, prefetch=True,
                     fire_gather=True)
            return carry

        lax.fori_loop(1, rounds - 1, round_body, 0)

        t0 = (rounds - 1) * _NBUF
        tail = nsteps - t0
        for i in range(tail):
            t = t0 + i
            p = i % _NBUF
            step(t, p,
                 drain_store=(t + 2 < nsteps),
                 prefetch=(t + 2 < nsteps),
                 fire_gather=(t + 1 < nsteps))
        for i in range(tail - _NBUF, tail):
            wait_store(i % _NBUF)

    return k(xT, labelsT, table)


def kernel(x, cluster_labels, table):
    b, lp1, d = x.shape
    xT = jnp.transpose(x, (1, 2, 0))
    labelsT = jnp.concatenate(
        [jnp.zeros((1, b), dtype=cluster_labels.dtype),
         cluster_labels.T], axis=0)
    outT = _sc_gather_add_t(xT, labelsT, table)
    return jnp.transpose(outT, (2, 0, 1))


# speedup vs baseline: 1.2365x; 1.2365x over previous
import functools

import jax
import jax.numpy as jnp
from jax import lax
from jax.experimental import pallas as pl
from jax.experimental.pallas import tpu as pltpu
from jax.experimental.pallas import tpu_sc as plsc

_NC = 2
_NS = 16
_NW = _NC * _NS
_L = 16
_NBUF = 4


def _sc_gather_add_t(x4, lab3, table):
    t_dim, dt, nb, tile = x4.shape
    v, d = table.shape
    bw = 128
    assert nb == _NW and tile == 1024 and d == 64

    mesh = plsc.VectorSubcoreMesh(
        core_axis_name="c", subcore_axis_name="s",
        num_cores=_NC, num_subcores=_NS)

    @functools.partial(
        pl.kernel,
        out_type=jax.ShapeDtypeStruct((t_dim, dt, nb, tile), jnp.float32),
        mesh=mesh,
        compiler_params=pltpu.CompilerParams(
            use_tc_tiling_on_sc=False, needs_layout_passes=False),
        scratch_types=(
            [pltpu.VMEM((bw,), jnp.int32) for _ in range(_NBUF)]
            + [pltpu.VMEM((bw, d), jnp.float32) for _ in range(_NBUF)]
            + [pltpu.VMEM((dt, tile), jnp.float32) for _ in range(_NBUF)]
            + [pltpu.SemaphoreType.DMA for _ in range(4 * _NBUF)]
        ),
    )
    def k(x_hbm, lab_hbm, tab_hbm, out_hbm, *scr):
        idx_v = scr[:_NBUF]
        g_v = scr[_NBUF:2 * _NBUF]
        xo_v = scr[2 * _NBUF:3 * _NBUF]
        s_ix = scr[3 * _NBUF:4 * _NBUF]
        s_x = scr[4 * _NBUF:5 * _NBUF]
        s_g = scr[5 * _NBUF:6 * _NBUF]
        s_o = scr[6 * _NBUF:7 * _NBUF]
        wid = lax.axis_index("s") * _NC + lax.axis_index("c")
        lane = lax.iota(jnp.int32, 16)

        def load(t, p):
            pltpu.async_copy(lab_hbm.at[t, wid], idx_v[p], s_ix[p])
            pltpu.async_copy(x_hbm.at[t, :, wid], xo_v[p], s_x[p])

        def wait_idx(p):
            pltpu.make_async_copy(lab_hbm.at[0, wid], idx_v[p], s_ix[p]).wait()

        def wait_x(p):
            pltpu.make_async_copy(x_hbm.at[0, :, wid], xo_v[p], s_x[p]).wait()

        def gather(p):
            pltpu.async_copy(tab_hbm.at[idx_v[p]], g_v[p], s_g[p])

        def wait_gather(p):
            pltpu.make_async_copy(tab_hbm.at[idx_v[p]], g_v[p], s_g[p]).wait()

        def store(t, p):
            pltpu.async_copy(xo_v[p], out_hbm.at[t, :, wid], s_o[p])

        def wait_store(p):
            pltpu.make_async_copy(xo_v[p], out_hbm.at[0, :, wid], s_o[p]).wait()

        def compute(p):
            g = g_v[p]
            xo = xo_v[p]
            d2n = d // dt

            def bg_body(bg, c2):
                bidx = bg * _L + lane
                for d1 in range(dt):
                    for d2 in range(d2n):
                        dd = d1 * d2n + d2
                        vals = plsc.load_gather(
                            g, [bidx, jnp.full((16,), dd, jnp.int32)])
                        plsc.addupdate(
                            xo.at[d1, pl.ds(d2 * 128 + bg * _L, _L)], vals)
                return c2

            lax.fori_loop(0, bw // _L, bg_body, 0)

        def step(t, p, drain_store, prefetch, fire_gather):
            q2 = (p + 2) % _NBUF
            if drain_store:
                wait_store(q2)
            if prefetch:
                load(t + 2, q2)
            q1 = (p + 1) % _NBUF
            if fire_gather:
                wait_idx(q1)
                gather(q1)
            wait_gather(p)
            wait_x(p)
            compute(p)
            store(t, p)

        nsteps = t_dim
        rounds = nsteps // _NBUF
        load(0, 0)
        load(1, 1)
        wait_idx(0)
        gather(0)

        for p in range(_NBUF):
            step(p, p, drain_store=(p >= 2), prefetch=True, fire_gather=True)

        def round_body(gr, carry):
            t0 = gr * _NBUF
            for p in range(_NBUF):
                step(t0 + p, p, drain_store=True, prefetch=True,
                     fire_gather=True)
            return carry

        lax.fori_loop(1, rounds - 1, round_body, 0)

        t0 = (rounds - 1) * _NBUF
        tail = nsteps - t0
        for i in range(tail):
            t = t0 + i
            step(t, i % _NBUF,
                 drain_store=(t + 2 < nsteps),
                 prefetch=(t + 2 < nsteps),
                 fire_gather=(t + 1 < nsteps))
        for i in range(tail - _NBUF, tail):
            wait_store(i % _NBUF)

    return k(x4, lab3, table)


def kernel(x, cluster_labels, table):
    b, lp1, d = x.shape
    x4 = (x.reshape(32, 128, lp1, 8, 8)
          .transpose(2, 3, 0, 4, 1)
          .reshape(lp1, 8, 32, 1024))
    labels = jnp.concatenate(
        [jnp.zeros((1, b), dtype=cluster_labels.dtype),
         cluster_labels.T], axis=0)
    lab3 = labels.reshape(lp1, 32, 128)
    out4 = _sc_gather_add_t(x4, lab3, table)
    return (out4.reshape(lp1, 8, 32, 8, 128)
            .transpose(2, 4, 0, 1, 3)
            .reshape(b, lp1, d))


# speedup vs baseline: 1.9308x; 1.5615x over previous
import functools

import jax
import jax.numpy as jnp
from jax import lax
from jax.experimental import pallas as pl
from jax.experimental.pallas import tpu as pltpu
from jax.experimental.pallas import tpu_sc as plsc

_NC = 2
_NS = 16
_NW = _NC * _NS
_L = 16
_NBUF = 4


def _sc_gather_add_t(x4, lab3, table):
    t_dim, dt, nb, tile = x4.shape
    v, d = table.shape
    bw = 128
    assert nb == _NW and tile == 1024 and d == 64

    mesh = plsc.VectorSubcoreMesh(
        core_axis_name="c", subcore_axis_name="s",
        num_cores=_NC, num_subcores=_NS)

    @functools.partial(
        pl.kernel,
        out_type=jax.ShapeDtypeStruct((t_dim, dt, nb, tile), jnp.float32),
        mesh=mesh,
        compiler_params=pltpu.CompilerParams(
            use_tc_tiling_on_sc=False, needs_layout_passes=False),
        scratch_types=(
            [pltpu.VMEM((bw,), jnp.int32) for _ in range(_NBUF)]
            + [pltpu.VMEM((bw, d), jnp.float32) for _ in range(_NBUF)]
            + [pltpu.VMEM((dt, tile), jnp.float32) for _ in range(_NBUF)]
            + [pltpu.SemaphoreType.DMA for _ in range(4 * _NBUF)]
        ),
    )
    def k(x_hbm, lab_hbm, tab_hbm, out_hbm, *scr):
        idx_v = scr[:_NBUF]
        g_v = scr[_NBUF:2 * _NBUF]
        xo_v = scr[2 * _NBUF:3 * _NBUF]
        s_ix = scr[3 * _NBUF:4 * _NBUF]
        s_x = scr[4 * _NBUF:5 * _NBUF]
        s_g = scr[5 * _NBUF:6 * _NBUF]
        s_o = scr[6 * _NBUF:7 * _NBUF]
        wid = lax.axis_index("s") * _NC + lax.axis_index("c")
        lane = lax.iota(jnp.int32, 16)

        def load(t, p):
            pltpu.async_copy(lab_hbm.at[t, wid], idx_v[p], s_ix[p])
            pltpu.async_copy(x_hbm.at[t, :, wid], xo_v[p], s_x[p])

        def wait_idx(p):
            pltpu.make_async_copy(lab_hbm.at[0, wid], idx_v[p], s_ix[p]).wait()

        def wait_x(p):
            pltpu.make_async_copy(x_hbm.at[0, :, wid], xo_v[p], s_x[p]).wait()

        def gather(p):
            pltpu.async_copy(tab_hbm.at[idx_v[p]], g_v[p], s_g[p])

        def wait_gather(p):
            pltpu.make_async_copy(tab_hbm.at[idx_v[p]], g_v[p], s_g[p]).wait()

        def store(t, p):
            pltpu.async_copy(xo_v[p], out_hbm.at[t, :, wid], s_o[p])

        def wait_store(p):
            pltpu.make_async_copy(xo_v[p], out_hbm.at[0, :, wid], s_o[p]).wait()

        def compute(p):
            g = g_v[p]
            xo = xo_v[p]
            d2n = d // dt

            for bg in range(bw // _L):
                bidx = bg * _L + lane

                @plsc.parallel_loop(0, d, 1, unroll=8)
                def dbody(dd, bg=bg, bidx=bidx, g=g, xo=xo):
                    d1 = dd // d2n
                    d2 = dd % d2n
                    vals = plsc.load_gather(
                        g, [bidx, jnp.full((16,), dd, jnp.int32)])
                    plsc.addupdate(
                        xo.at[d1, pl.ds(d2 * 128 + bg * _L, _L)], vals)

        def step(t, p, drain_store, prefetch, fire_gather):
            q2 = (p + 2) % _NBUF
            if drain_store:
                wait_store(q2)
            if prefetch:
                load(t + 2, q2)
            q1 = (p + 1) % _NBUF
            if fire_gather:
                wait_idx(q1)
                gather(q1)
            wait_gather(p)
            wait_x(p)
            compute(p)
            store(t, p)

        nsteps = t_dim
        rounds = nsteps // _NBUF
        load(0, 0)
        load(1, 1)
        wait_idx(0)
        gather(0)

        for p in range(_NBUF):
            step(p, p, drain_store=(p >= 2), prefetch=True, fire_gather=True)

        def round_body(gr, carry):
            t0 = gr * _NBUF
            for p in range(_NBUF):
                step(t0 + p, p, drain_store=True, prefetch=True,
                     fire_gather=True)
            return carry

        lax.fori_loop(1, rounds - 1, round_body, 0)

        t0 = (rounds - 1) * _NBUF
        tail = nsteps - t0
        for i in range(tail):
            t = t0 + i
            step(t, i % _NBUF,
                 drain_store=(t + 2 < nsteps),
                 prefetch=(t + 2 < nsteps),
                 fire_gather=(t + 1 < nsteps))
        for i in range(tail - _NBUF, tail):
            wait_store(i % _NBUF)

    return k(x4, lab3, table)


def kernel(x, cluster_labels, table):
    b, lp1, d = x.shape
    x4 = (x.reshape(32, 128, lp1, 8, 8)
          .transpose(2, 3, 0, 4, 1)
          .reshape(lp1, 8, 32, 1024))
    labels = jnp.concatenate(
        [jnp.zeros((1, b), dtype=cluster_labels.dtype),
         cluster_labels.T], axis=0)
    lab3 = labels.reshape(lp1, 32, 128)
    out4 = _sc_gather_add_t(x4, lab3, table)
    return (out4.reshape(lp1, 8, 32, 8, 128)
            .transpose(2, 4, 0, 1, 3)
            .reshape(b, lp1, d))


# speedup vs baseline: 2.0061x; 1.0390x over previous
import functools

import jax
import jax.numpy as jnp
from jax import lax
from jax.experimental import pallas as pl
from jax.experimental.pallas import tpu as pltpu
from jax.experimental.pallas import tpu_sc as plsc

_NC = 2
_NS = 16
_NW = _NC * _NS
_L = 16
_NBUF = 6


def _sc_gather_add_t(x4, lab3, table):
    t_dim, dt, nb, tile = x4.shape
    v, d = table.shape
    bw = 128
    assert nb == _NW and tile == 1024 and d == 64

    mesh = plsc.VectorSubcoreMesh(
        core_axis_name="c", subcore_axis_name="s",
        num_cores=_NC, num_subcores=_NS)

    @functools.partial(
        pl.kernel,
        out_type=jax.ShapeDtypeStruct((t_dim, dt, nb, tile), jnp.float32),
        mesh=mesh,
        compiler_params=pltpu.CompilerParams(
            use_tc_tiling_on_sc=False, needs_layout_passes=False),
        scratch_types=(
            [pltpu.VMEM((t_dim, bw), jnp.int32)]
            + [pltpu.VMEM((bw, d), jnp.float32) for _ in range(_NBUF)]
            + [pltpu.VMEM((dt, tile), jnp.float32) for _ in range(_NBUF)]
            + [pltpu.SemaphoreType.DMA for _ in range(3 * _NBUF)]
        ),
    )
    def k(x_hbm, lab_hbm, tab_hbm, out_hbm, *scr):
        idx_all = scr[0]
        g_v = scr[1:1 + _NBUF]
        xo_v = scr[1 + _NBUF:1 + 2 * _NBUF]
        s_x = scr[1 + 2 * _NBUF:1 + 3 * _NBUF]
        s_g = scr[1 + 3 * _NBUF:1 + 4 * _NBUF]
        s_o = scr[1 + 4 * _NBUF:1 + 5 * _NBUF]
        wid = lax.axis_index("s") * _NC + lax.axis_index("c")
        lane = lax.iota(jnp.int32, 16)

        def load(t, p):
            pltpu.async_copy(x_hbm.at[t, :, wid], xo_v[p], s_x[p])

        def wait_x(p):
            pltpu.make_async_copy(x_hbm.at[0, :, wid], xo_v[p], s_x[p]).wait()

        def gather(t, p):
            pltpu.async_copy(tab_hbm.at[idx_all.at[t]], g_v[p], s_g[p])

        def wait_gather(p):
            pltpu.make_async_copy(
                tab_hbm.at[idx_all.at[0]], g_v[p], s_g[p]).wait()

        def store(t, p):
            pltpu.async_copy(xo_v[p], out_hbm.at[t, :, wid], s_o[p])

        def wait_store(p):
            pltpu.make_async_copy(xo_v[p], out_hbm.at[0, :, wid], s_o[p]).wait()

        def compute(p):
            g = g_v[p]
            xo = xo_v[p]
            d2n = d // dt

            @plsc.parallel_loop(0, (bw // _L) * d, 1, unroll=8)
            def body(i, g=g, xo=xo):
                bg = i // d
                dd = i % d
                d1 = dd // d2n
                d2 = dd % d2n
                vals = plsc.load_gather(
                    g, [bg * _L + lane, jnp.full((16,), dd, jnp.int32)])
                plsc.addupdate(
                    xo.at[d1, pl.ds(d2 * 128 + bg * _L, _L)], vals)

        def step(t, p, drain_store, prefetch, fire_gather):
            q3 = (p + 3) % _NBUF
            if drain_store:
                wait_store(q3)
            if prefetch:
                load(t + 3, q3)
            if fire_gather:
                gather(t + 2, (p + 2) % _NBUF)
            wait_gather(p)
            wait_x(p)
            compute(p)
            store(t, p)

        nsteps = t_dim
        rounds = nsteps // _NBUF
        pltpu.async_copy(lab_hbm.at[:, wid, :], idx_all, s_x[0])
        pltpu.make_async_copy(lab_hbm.at[:, wid, :], idx_all, s_x[0]).wait()
        load(0, 0)
        load(1, 1)
        load(2, 2)
        gather(0, 0)
        gather(1, 1)

        for p in range(_NBUF):
            step(p, p, drain_store=(p >= 3), prefetch=True, fire_gather=True)

        def round_body(gr, carry):
            t0 = gr * _NBUF
            for p in range(_NBUF):
                step(t0 + p, p, drain_store=True, prefetch=True,
                     fire_gather=True)
            return carry

        lax.fori_loop(1, rounds - 1, round_body, 0)

        t0 = (rounds - 1) * _NBUF
        tail = nsteps - t0
        for i in range(tail):
            t = t0 + i
            step(t, (t0 + i) % _NBUF,
                 drain_store=(t + 3 < nsteps),
                 prefetch=(t + 3 < nsteps),
                 fire_gather=(t + 2 < nsteps))
        for i in range(tail - _NBUF, tail):
            wait_store((t0 + i) % _NBUF)

    return k(x4, lab3, table)


def kernel(x, cluster_labels, table):
    b, lp1, d = x.shape
    x4 = (x.reshape(32, 128, lp1, 8, 8)
          .transpose(2, 3, 0, 4, 1)
          .reshape(lp1, 8, 32, 1024))
    labels = jnp.concatenate(
        [jnp.zeros((1, b), dtype=cluster_labels.dtype),
         cluster_labels.T], axis=0)
    lab3 = labels.reshape(lp1, 32, 128)
    out4 = _sc_gather_add_t(x4, lab3, table)
    return (out4.reshape(lp1, 8, 32, 8, 128)
            .transpose(2, 4, 0, 1, 3)
            .reshape(b, lp1, d))
